# baseline (speedup 1.0000x reference)
import functools

import jax
import jax.numpy as jnp
from jax import lax
from jax.experimental import pallas as pl
from jax.experimental.pallas import tpu as pltpu
from jax.experimental.pallas import tpu_sc as plsc

EPS = 1e-5

_INFO = plsc.get_sparse_core_info()
_NC = _INFO.num_cores
_NS = _INFO.num_subcores
_NW = _NC * _NS


def _sc_gather(user_table, user_id2d, item_table, item_id2d, B, E):
    b_per_w = B // _NW
    n_chunks = b_per_w // 128
    mesh = plsc.VectorSubcoreMesh(core_axis_name="c", subcore_axis_name="s")

    @functools.partial(
        pl.kernel,
        mesh=mesh,
        compiler_params=pltpu.CompilerParams(use_tc_tiling_on_sc=False),
        out_type=[
            jax.ShapeDtypeStruct((B, E), jnp.float32),
            jax.ShapeDtypeStruct((B, E), jnp.float32),
        ],
        scratch_types=[
            pltpu.VMEM((n_chunks, 128), jnp.int32),
            pltpu.VMEM((b_per_w, E), jnp.float32),
            pltpu.VMEM((n_chunks, 128), jnp.int32),
            pltpu.VMEM((b_per_w, E), jnp.float32),
            pltpu.SemaphoreType.DMA,
        ],
    )
    def k(ut, uid, it, iid, out_u, out_i, uidx_v, urows_v, iidx_v, irows_v, sem):
        wid = lax.axis_index("s") * _NC + lax.axis_index("c")
        row0 = wid * n_chunks
        pltpu.sync_copy(uid.at[pl.ds(row0, n_chunks)], uidx_v)
        pltpu.sync_copy(iid.at[pl.ds(row0, n_chunks)], iidx_v)
        copies = []
        for j in range(n_chunks):
            copies.append(
                pltpu.async_copy(ut.at[uidx_v.at[j]],
                                 urows_v.at[pl.ds(j * 128, 128)], sem))
            copies.append(
                pltpu.async_copy(it.at[iidx_v.at[j]],
                                 irows_v.at[pl.ds(j * 128, 128)], sem))
        for c in copies:
            c.wait()
        base = wid * b_per_w
        pltpu.sync_copy(urows_v, out_u.at[pl.ds(base, b_per_w)])
        pltpu.sync_copy(irows_v, out_i.at[pl.ds(base, b_per_w)])

    return k(user_table, user_id2d, item_table, item_id2d)


def _tc_body(eu_ref, tu_ref, ei_ref, ti_ref,
             uofa_ref, uwo_ref, uwe_ref, uwf_ref,
             iofa_ref, iwo_ref, iwe_ref, iwf_ref,
             hu_ref, hi_ref):
    hp = jax.lax.Precision.HIGHEST

    bias_u = jnp.dot(uofa_ref[...], uwo_ref[...], precision=hp)
    hu = (jnp.dot(eu_ref[...], uwe_ref[...], precision=hp)
          + jnp.dot(tu_ref[...], uwf_ref[...], precision=hp)
          + bias_u)
    su = jnp.sum(hu * hu, axis=1, keepdims=True)
    hu_ref[...] = hu / (jnp.sqrt(su) + EPS)

    bias_i = jnp.dot(iofa_ref[...], iwo_ref[...], precision=hp)
    hi = (jnp.dot(ei_ref[...], iwe_ref[...], precision=hp)
          + jnp.dot(ti_ref[...], iwf_ref[...], precision=hp)
          + bias_i)
    si = jnp.sum(hi * hi, axis=1, keepdims=True)
    hi_ref[...] = hi / (jnp.sqrt(si) + EPS)


def _tc_fused(emb_u, t_users, emb_i, t_items,
              uofa, uwo, uwe, uwf, iofa, iwo, iwe, iwf, B, HID):
    bM = 2048
    grid = (B // bM,)
    row = lambda i: (i, 0)
    rep = lambda i: (0, 0)
    return pl.pallas_call(
        _tc_body,
        grid=grid,
        in_specs=[
            pl.BlockSpec((bM, emb_u.shape[1]), row),
            pl.BlockSpec((bM, t_users.shape[1]), row),
            pl.BlockSpec((bM, emb_i.shape[1]), row),
            pl.BlockSpec((bM, t_items.shape[1]), row),
            pl.BlockSpec(uofa.shape, rep),
            pl.BlockSpec(uwo.shape, rep),
            pl.BlockSpec(uwe.shape, rep),
            pl.BlockSpec(uwf.shape, rep),
            pl.BlockSpec(iofa.shape, rep),
            pl.BlockSpec(iwo.shape, rep),
            pl.BlockSpec(iwe.shape, rep),
            pl.BlockSpec(iwf.shape, rep),
        ],
        out_specs=[
            pl.BlockSpec((bM, HID), row),
            pl.BlockSpec((bM, HID), row),
        ],
        out_shape=[
            jax.ShapeDtypeStruct((B, HID), jnp.float32),
            jax.ShapeDtypeStruct((B, HID), jnp.float32),
        ],
    )(emb_u, t_users, emb_i, t_items,
      uofa, uwo, uwe, uwf, iofa, iwo, iwe, iwf)


@jax.jit
def kernel(t_users, user_id, t_items, item_id, user_ofa, user_table, user_W,
           item_ofa, item_table, item_W):
    B = user_id.shape[0]
    E = user_table.shape[1]
    HID = user_W.shape[0]
    OFA = user_ofa.shape[1]
    FVS = t_users.shape[1]

    uid2d = user_id.reshape(B // 128, 128)
    iid2d = item_id.reshape(B // 128, 128)
    emb_u, emb_i = _sc_gather(user_table, uid2d, item_table, iid2d, B, E)

    uwo = user_W[:, :OFA].T
    uwe = user_W[:, OFA:OFA + E].T
    uwf = user_W[:, OFA + E:].T
    iwo = item_W[:, :OFA].T
    iwe = item_W[:, OFA:OFA + E].T
    iwf = item_W[:, OFA + E:].T

    h_user, h_item = _tc_fused(emb_u, t_users, emb_i, t_items,
                               user_ofa, uwo, uwe, uwf,
                               item_ofa, iwo, iwe, iwf, B, HID)
    return (h_user, h_item)
